# baseline (speedup 1.0000x reference)
import jax
import jax.numpy as jnp
from jax import lax
from jax.experimental import pallas as pl
from jax.experimental.pallas import tpu as pltpu

_TAU = 1.0
_TILE_B = 2048
_SPLIT = 16
_SUB = _TILE_B // _SPLIT


def _fused_scorer_kernel(*refs):
    (*i_refs, wi_ref, wrt_ref, a_ref, probs_ref, logits_ref) = refs
    proj_a = lax.dot_general(
        a_ref[...], wrt_ref[...], (((1,), (0,)), ((), ())),
        preferred_element_type=jnp.float32)
    proj_i = jnp.concatenate(
        [lax.dot_general(ref[...], wi_ref[...], (((1,), (1,)), ((), ())),
                         preferred_element_type=jnp.float32)
         for ref in i_refs],
        axis=0)
    logits = lax.dot_general(
        proj_i, proj_a, (((1,), (1,)), ((), ())),
        preferred_element_type=jnp.float32)

    k = logits.shape[1]
    lt_ = logits.T
    logits_ref[...] = lt_
    lt = (lax.broadcasted_iota(jnp.int32, (k, k), 1)
          <= lax.broadcasted_iota(jnp.int32, (k, k), 0)).astype(jnp.float32)
    m1 = jnp.max(lt_, axis=0, keepdims=True)
    is1 = (lt_ == m1).astype(jnp.float32)
    c1 = lax.dot_general(lt, is1, (((1,), (0,)), ((), ())),
                         preferred_element_type=jnp.float32)
    first1 = (is1 > 0.0) & (c1 == 1.0)
    l2 = jnp.where(first1, -jnp.inf, lt_)
    m2 = jnp.max(l2, axis=0, keepdims=True)
    is2 = (l2 == m2).astype(jnp.float32)
    c2 = lax.dot_general(lt, is2, (((1,), (0,)), ((), ())),
                         preferred_element_type=jnp.float32)
    first2 = (is2 > 0.0) & (c2 == 1.0)
    r = jnp.exp((m2 - m1) / _TAU)
    denom = 1.0 + r
    probs_ref[...] = jnp.where(first1, 1.0 / denom,
                               jnp.where(first2, r / denom, 0.0))


def _i_spec(j):
    return pl.BlockSpec((_SUB, 768), lambda i, j=j: (_SPLIT * i + j, 0))


def kernel(I, Wi, Wr, A):
    B, d_in = I.shape
    k = A.shape[0]
    out = pl.pallas_call(
        _fused_scorer_kernel,
        grid=(B // _TILE_B,),
        in_specs=[
            *[_i_spec(j) for j in range(_SPLIT)],
            pl.BlockSpec(Wi.shape, lambda i: (0, 0)),
            pl.BlockSpec((k, Wr.shape[0]), lambda i: (0, 0)),
            pl.BlockSpec(A.shape, lambda i: (0, 0)),
        ],
        out_specs=[
            pl.BlockSpec((k, _TILE_B), lambda i: (0, i)),
            pl.BlockSpec((k, _TILE_B), lambda i: (0, i)),
        ],
        out_shape=[
            jax.ShapeDtypeStruct((k, B), jnp.float32),
            jax.ShapeDtypeStruct((k, B), jnp.float32),
        ],
        compiler_params=pltpu.CompilerParams(
            dimension_semantics=("parallel",)),
    )(*([I] * _SPLIT), Wi, Wr.T, A)
    return (out[0].T, out[1].T)


# speedup vs baseline: 1.1369x; 1.1369x over previous
import jax
import jax.numpy as jnp
from jax import lax
from jax.experimental import pallas as pl
from jax.experimental.pallas import tpu as pltpu

_TAU = 1.0
_TILE_B = 4096
_SPLIT = 16
_SUB = _TILE_B // _SPLIT


def _fused_scorer_kernel(*refs):
    (*i_refs, wi_ref, wrt_ref, a_ref, probs_ref, logits_ref) = refs
    proj_a = lax.dot_general(
        a_ref[...], wrt_ref[...], (((1,), (0,)), ((), ())),
        preferred_element_type=jnp.float32)
    proj_i = jnp.concatenate(
        [lax.dot_general(ref[...], wi_ref[...], (((1,), (1,)), ((), ())),
                         preferred_element_type=jnp.float32)
         for ref in i_refs],
        axis=0)
    logits = lax.dot_general(
        proj_i, proj_a, (((1,), (1,)), ((), ())),
        preferred_element_type=jnp.float32)

    k = logits.shape[1]
    lt_ = logits.T
    logits_ref[...] = lt_
    lt = (lax.broadcasted_iota(jnp.int32, (k, k), 1)
          <= lax.broadcasted_iota(jnp.int32, (k, k), 0)).astype(jnp.float32)
    m1 = jnp.max(lt_, axis=0, keepdims=True)
    is1 = (lt_ == m1).astype(jnp.float32)
    c1 = lax.dot_general(lt, is1, (((1,), (0,)), ((), ())),
                         preferred_element_type=jnp.float32)
    first1 = (is1 > 0.0) & (c1 == 1.0)
    l2 = jnp.where(first1, -jnp.inf, lt_)
    m2 = jnp.max(l2, axis=0, keepdims=True)
    is2 = (l2 == m2).astype(jnp.float32)
    c2 = lax.dot_general(lt, is2, (((1,), (0,)), ((), ())),
                         preferred_element_type=jnp.float32)
    first2 = (is2 > 0.0) & (c2 == 1.0)
    r = jnp.exp((m2 - m1) / _TAU)
    denom = 1.0 + r
    probs_ref[...] = jnp.where(first1, 1.0 / denom,
                               jnp.where(first2, r / denom, 0.0))


def _i_spec(j):
    return pl.BlockSpec((_SUB, 768), lambda i, j=j: (_SPLIT * i + j, 0))


def kernel(I, Wi, Wr, A):
    B, d_in = I.shape
    k = A.shape[0]
    out = pl.pallas_call(
        _fused_scorer_kernel,
        grid=(B // _TILE_B,),
        in_specs=[
            *[_i_spec(j) for j in range(_SPLIT)],
            pl.BlockSpec(Wi.shape, lambda i: (0, 0)),
            pl.BlockSpec((k, Wr.shape[0]), lambda i: (0, 0)),
            pl.BlockSpec(A.shape, lambda i: (0, 0)),
        ],
        out_specs=[
            pl.BlockSpec((k, _TILE_B), lambda i: (0, i)),
            pl.BlockSpec((k, _TILE_B), lambda i: (0, i)),
        ],
        out_shape=[
            jax.ShapeDtypeStruct((k, B), jnp.float32),
            jax.ShapeDtypeStruct((k, B), jnp.float32),
        ],
        compiler_params=pltpu.CompilerParams(
            dimension_semantics=("parallel",)),
    )(*([I] * _SPLIT), Wi, Wr.T, A)
    return (out[0].T, out[1].T)
